# baseline (device time: 133348 ns/iter reference)
import jax
import jax.numpy as jnp
from jax import lax
from jax.experimental import pallas as pl
from jax.experimental.pallas import tpu as pltpu

T = 2048
D = 4096
V_SHARD = 8192
V_HALF = V_SHARD // 2
V_TILE = 512
N_TILES = V_HALF // V_TILE


def _stats_body(y_sref, x_ref, w_ref, labels_ref, stats_ref):
    i = pl.program_id(0)
    my_x = lax.axis_index("x")
    my_y = lax.axis_index("y")

    logits = jnp.dot(
        x_ref[:, :], w_ref[:, :], preferred_element_type=jnp.float32
    )

    labs_shift = labels_ref[:, :] - (my_x * V_SHARD + my_y * V_HALF + i * V_TILE)
    cols = lax.broadcasted_iota(jnp.int32, (T, V_TILE), 1)
    lmask = jnp.where(cols == labs_shift, logits, 0.0)

    ones = jnp.ones((V_TILE, 128), jnp.float32)
    sumexp = jnp.dot(jnp.exp(logits), ones, preferred_element_type=jnp.float32)[
        :, 0:1
    ]
    lab = jnp.dot(lmask, ones, preferred_element_type=jnp.float32)[:, 0:1]

    @pl.when(i == 0)
    def _():
        stats_ref[:, 0:1] = sumexp
        stats_ref[:, 1:2] = lab

    @pl.when(i != 0)
    def _():
        stats_ref[:, 0:1] += sumexp
        stats_ref[:, 1:2] += lab


def _combine_body(
    stats_ref, out_ref, comm1_ref, acc_ref, comm2_ref, s1_send, s1_recv, s2_send, s2_recv
):
    my_x = lax.axis_index("x")
    my_y = lax.axis_index("y")
    xpeer = (1 - my_x, my_y)
    ypeer = (my_x, 1 - my_y)

    barrier = pltpu.get_barrier_semaphore()
    pl.semaphore_signal(
        barrier, inc=1, device_id=xpeer, device_id_type=pl.DeviceIdType.MESH
    )
    pl.semaphore_signal(
        barrier, inc=1, device_id=ypeer, device_id_type=pl.DeviceIdType.MESH
    )
    pl.semaphore_wait(barrier, 2)

    rdma1 = pltpu.make_async_remote_copy(
        src_ref=stats_ref,
        dst_ref=comm1_ref,
        send_sem=s1_send,
        recv_sem=s1_recv,
        device_id=xpeer,
        device_id_type=pl.DeviceIdType.MESH,
    )
    rdma1.start()
    rdma1.wait()

    acc_ref[:, :] = stats_ref[:, :] + comm1_ref[:, :]

    rdma2 = pltpu.make_async_remote_copy(
        src_ref=acc_ref,
        dst_ref=comm2_ref,
        send_sem=s2_send,
        recv_sem=s2_recv,
        device_id=ypeer,
        device_id_type=pl.DeviceIdType.MESH,
    )
    rdma2.start()
    rdma2.wait()

    s = acc_ref[:, 0:1] + comm2_ref[:, 0:1]
    lab = acc_ref[:, 1:2] + comm2_ref[:, 1:2]
    out_ref[:, :] = jnp.log(s) - lab


def kernel(x, W, labels):
    labels2 = labels.reshape(T, 1).astype(jnp.int32)
    my_y = lax.axis_index("y").reshape(1).astype(jnp.int32)

    stats = pl.pallas_call(
        _stats_body,
        grid_spec=pltpu.PrefetchScalarGridSpec(
            num_scalar_prefetch=1,
            grid=(N_TILES,),
            in_specs=[
                pl.BlockSpec((T, D), lambda i, y: (0, 0)),
                pl.BlockSpec((D, V_TILE), lambda i, y: (0, y[0] * N_TILES + i)),
                pl.BlockSpec((T, 1), lambda i, y: (0, 0)),
            ],
            out_specs=pl.BlockSpec((T, 2), lambda i, y: (0, 0)),
        ),
        out_shape=jax.ShapeDtypeStruct((T, 2), jnp.float32),
        compiler_params=pltpu.CompilerParams(
            dimension_semantics=("arbitrary",),
            vmem_limit_bytes=100 * 1024 * 1024,
        ),
    )(my_y, x, W, labels2)

    nll = pl.pallas_call(
        _combine_body,
        in_specs=[pl.BlockSpec(memory_space=pltpu.VMEM)],
        out_specs=pl.BlockSpec(memory_space=pltpu.VMEM),
        out_shape=jax.ShapeDtypeStruct((T, 1), jnp.float32),
        scratch_shapes=[
            pltpu.VMEM((T, 2), jnp.float32),
            pltpu.VMEM((T, 2), jnp.float32),
            pltpu.VMEM((T, 2), jnp.float32),
            pltpu.SemaphoreType.DMA,
            pltpu.SemaphoreType.DMA,
            pltpu.SemaphoreType.DMA,
            pltpu.SemaphoreType.DMA,
        ],
        compiler_params=pltpu.CompilerParams(collective_id=0),
    )(stats)

    return nll[:, 0]


# device time: 103229 ns/iter; 1.2918x vs baseline; 1.2918x over previous
import jax
import jax.numpy as jnp
from jax import lax
from jax.experimental import pallas as pl
from jax.experimental.pallas import tpu as pltpu

T = 2048
D = 4096
V_SHARD = 8192
V_HALF = V_SHARD // 2
V_TILE = 512
N_TILES = V_HALF // V_TILE


def _stats_body(y_sref, x_ref, w_ref, labels_ref, stats_ref):
    i = pl.program_id(0)
    my_x = lax.axis_index("x")
    my_y = lax.axis_index("y")

    logits = jnp.dot(
        x_ref[:, :], w_ref[:, :], preferred_element_type=jnp.float32
    )

    labs_shift = labels_ref[:, :] - (my_x * V_SHARD + my_y * V_HALF + i * V_TILE)
    cols = lax.broadcasted_iota(jnp.int32, (T, V_TILE), 1)
    sumexp = jnp.sum(jnp.exp(logits), axis=1, keepdims=True)
    lab = jnp.sum(
        jnp.where(cols == labs_shift, logits, 0.0), axis=1, keepdims=True
    )

    @pl.when(i == 0)
    def _():
        stats_ref[:, 0:1] = sumexp
        stats_ref[:, 1:2] = lab

    @pl.when(i != 0)
    def _():
        stats_ref[:, 0:1] += sumexp
        stats_ref[:, 1:2] += lab


def _combine_body(stats_ref, out_ref, comm_ref, send_sems, recv_sems):
    my_x = lax.axis_index("x")
    my_y = lax.axis_index("y")
    peers = [(1 - my_x, my_y), (my_x, 1 - my_y), (1 - my_x, 1 - my_y)]

    barrier = pltpu.get_barrier_semaphore()
    for p in peers:
        pl.semaphore_signal(
            barrier, inc=1, device_id=p, device_id_type=pl.DeviceIdType.MESH
        )
    pl.semaphore_wait(barrier, 3)

    rdmas = []
    for k, p in enumerate(peers):
        rdma = pltpu.make_async_remote_copy(
            src_ref=stats_ref,
            dst_ref=comm_ref.at[k],
            send_sem=send_sems.at[k],
            recv_sem=recv_sems.at[k],
            device_id=p,
            device_id_type=pl.DeviceIdType.MESH,
        )
        rdma.start()
        rdmas.append(rdma)
    for rdma in rdmas:
        rdma.wait()

    tot = (
        stats_ref[:, :] + comm_ref[0, :, :] + comm_ref[1, :, :] + comm_ref[2, :, :]
    )
    out_ref[:, :] = jnp.log(tot[0:1, :]) - tot[1:2, :]


def kernel(x, W, labels):
    labels2 = labels.reshape(T, 1).astype(jnp.int32)
    my_y = lax.axis_index("y").reshape(1).astype(jnp.int32)

    stats = pl.pallas_call(
        _stats_body,
        grid_spec=pltpu.PrefetchScalarGridSpec(
            num_scalar_prefetch=1,
            grid=(N_TILES,),
            in_specs=[
                pl.BlockSpec((T, D), lambda i, y: (0, 0)),
                pl.BlockSpec((D, V_TILE), lambda i, y: (0, y[0] * N_TILES + i)),
                pl.BlockSpec((T, 1), lambda i, y: (0, 0)),
            ],
            out_specs=pl.BlockSpec((T, 2), lambda i, y: (0, 0)),
        ),
        out_shape=jax.ShapeDtypeStruct((T, 2), jnp.float32),
        compiler_params=pltpu.CompilerParams(
            dimension_semantics=("arbitrary",),
            vmem_limit_bytes=100 * 1024 * 1024,
        ),
    )(my_y, x, W, labels2)

    stats_t = stats.T

    nll = pl.pallas_call(
        _combine_body,
        in_specs=[pl.BlockSpec(memory_space=pltpu.VMEM)],
        out_specs=pl.BlockSpec(memory_space=pltpu.VMEM),
        out_shape=jax.ShapeDtypeStruct((1, T), jnp.float32),
        scratch_shapes=[
            pltpu.VMEM((3, 2, T), jnp.float32),
            pltpu.SemaphoreType.DMA((3,)),
            pltpu.SemaphoreType.DMA((3,)),
        ],
        compiler_params=pltpu.CompilerParams(collective_id=0),
    )(stats_t)

    return nll[0]
